# baseline (speedup 1.0000x reference)
import functools

import jax
import jax.numpy as jnp
from jax.experimental import pallas as pl
from jax.experimental.pallas import tpu as pltpu

_ROWS = 64
_NCOLS = 1_000_000
_W = 8192
_SUB = 512
_NSUB = _W // _SUB
_GRID = (_NCOLS + _W - 1) // _W

_KS1 = 1234
_KS2 = 1234 ^ 0x1BD11BDA
_M32 = 0xFFFFFFFF
_INJ = (
    (_KS1, (_KS2 + 1) & _M32),
    (_KS2, 2),
    (0, _KS1 + 3),
    (_KS1, (_KS2 + 4) & _M32),
    (_KS2, 5),
)
_ROT = ((13, 15, 26, 6), (17, 29, 16, 24))


def _rotl(x, d):
    return (x << d) | (x >> (32 - d))


def _threefry_bits(a):
    x0 = a
    x1 = _rotl(a, 13) ^ a
    for d in (15, 26, 6):
        x0 = x0 + x1
        x1 = _rotl(x1, d) ^ x0
    x0 = x0 + jnp.uint32(_INJ[0][0])
    x1 = x1 + jnp.uint32(_INJ[0][1])
    for g in (1, 2, 3, 4):
        for d in _ROT[g % 2]:
            x0 = x0 + x1
            x1 = _rotl(x1, d) ^ x0
        if _INJ[g][0]:
            x0 = x0 + jnp.uint32(_INJ[g][0])
        x1 = x1 + jnp.uint32(_INJ[g][1])
    return x0 ^ x1


def _gumbel_val(bits, s):
    fb = (bits >> 9) | jnp.uint32(0x3F800000)
    u = jax.lax.bitcast_convert_type(fb, jnp.float32) - 1.0
    noise = jnp.maximum(-jnp.log1p(-u), 1e-10)
    return s - jnp.log(noise)


def _body(logits_ref, temp_ref, idx_ref, vmax_ref, va_ref):
    j = pl.program_id(0)
    rtemp = 1.0 / temp_ref[...]

    lane = jax.lax.broadcasted_iota(jnp.int32, (_ROWS, _SUB), 1)
    rowoff = jax.lax.broadcasted_iota(jnp.int32, (_ROWS, _SUB), 0) * _NCOLS
    a0 = (rowoff + lane + j * _W + _KS1).astype(jnp.uint32)

    vmax0 = jnp.where(j == 0, jnp.full((_ROWS, _SUB), -jnp.inf, jnp.float32),
                      vmax_ref[...])
    va0 = jnp.where(j == 0, jnp.zeros((_ROWS, _SUB), jnp.uint32),
                    va_ref[...])

    def sub(k, carry, masked):
        vmax, va, a = carry
        bits = _threefry_bits(a)
        s = logits_ref[:, pl.ds(k * _SUB, _SUB)] * rtemp
        val = _gumbel_val(bits, s)
        if masked:
            val = jnp.where(a < bound, val, -jnp.inf)
        upd = val > vmax
        return (jnp.where(upd, val, vmax), jnp.where(upd, a, va),
                a + jnp.uint32(_SUB))

    @pl.when(j < _GRID - 1)
    def _():
        vmax1, va1, _ = jax.lax.fori_loop(
            0, _NSUB, lambda k, c: sub(k, c, False), (vmax0, va0, a0))
        vmax_ref[...] = vmax1
        va_ref[...] = va1

    bound = (rowoff + (_NCOLS + _KS1)).astype(jnp.uint32)

    @pl.when(j == _GRID - 1)
    def _():
        vmax1, va1, _ = jax.lax.fori_loop(
            0, _NSUB, lambda k, c: sub(k, c, True), (vmax0, va0, a0))
        rmax = jnp.max(vmax1, axis=1, keepdims=True)
        col = (va1.astype(jnp.int32) - _KS1) - rowoff
        cand = jnp.where(vmax1 == rmax, col, jnp.int32(2**31 - 1))
        idx_ref[...] = jnp.min(cand, axis=1, keepdims=True)


@functools.partial(jax.jit, static_argnames=("interpret",))
def _sample(logits, temperature, interpret=False):
    idx = pl.pallas_call(
        _body,
        grid=(_GRID,),
        in_specs=[
            pl.BlockSpec((_ROWS, _W), lambda j: (0, j)),
            pl.BlockSpec((_ROWS, 1), lambda j: (0, 0)),
        ],
        out_specs=pl.BlockSpec((_ROWS, 1), lambda j: (0, 0)),
        out_shape=jax.ShapeDtypeStruct((_ROWS, 1), jnp.int32),
        scratch_shapes=[
            pltpu.VMEM((_ROWS, _SUB), jnp.float32),
            pltpu.VMEM((_ROWS, _SUB), jnp.uint32),
        ],
        interpret=interpret,
    )(logits, temperature.reshape(_ROWS, 1))
    return idx[:, 0]


def kernel(logits, temperature):
    return _sample(logits, temperature)


# speedup vs baseline: 1.2020x; 1.2020x over previous
import functools

import jax
import jax.numpy as jnp
from jax.experimental import pallas as pl
from jax.experimental.pallas import tpu as pltpu

_ROWS = 64
_NCOLS = 1_000_000
_W = 8192
_SUB = 256
_NSUB = _W // _SUB
_GRID = (_NCOLS + _W - 1) // _W

_KS1 = 1234
_KS2 = 1234 ^ 0x1BD11BDA
_M32 = 0xFFFFFFFF
_INJ = (
    (_KS1, (_KS2 + 1) & _M32),
    (_KS2, 2),
    (0, _KS1 + 3),
    (_KS1, (_KS2 + 4) & _M32),
    (_KS2, 5),
)
_ROT = ((13, 15, 26, 6), (17, 29, 16, 24))


def _rotl(x, d):
    return (x << d) | (x >> (32 - d))


def _threefry_bits(a):
    x0 = a
    x1 = _rotl(a, 13) ^ a
    for d in (15, 26, 6):
        x0 = x0 + x1
        x1 = _rotl(x1, d) ^ x0
    x0 = x0 + jnp.uint32(_INJ[0][0])
    x1 = x1 + jnp.uint32(_INJ[0][1])
    for g in (1, 2, 3, 4):
        for d in _ROT[g % 2]:
            x0 = x0 + x1
            x1 = _rotl(x1, d) ^ x0
        if _INJ[g][0]:
            x0 = x0 + jnp.uint32(_INJ[g][0])
        x1 = x1 + jnp.uint32(_INJ[g][1])
    return x0 ^ x1


def _gumbel_val(bits, s):
    fb = (bits >> 9) | jnp.uint32(0x3F800000)
    u = jax.lax.bitcast_convert_type(fb, jnp.float32) - 1.0
    noise = jnp.maximum(-jnp.log1p(-u), 1e-10)
    return s - jnp.log(noise)


def _body(logits_ref, temp_ref, idx_ref, vmax_ref, va_ref):
    j = pl.program_id(0)
    rtemp = 1.0 / temp_ref[...]

    lane = jax.lax.broadcasted_iota(jnp.int32, (_ROWS, _SUB), 1)
    rowoff = jax.lax.broadcasted_iota(jnp.int32, (_ROWS, _SUB), 0) * _NCOLS
    a0 = (rowoff + lane + j * _W + _KS1).astype(jnp.uint32)

    vmax0 = jnp.where(j == 0, jnp.full((_ROWS, _SUB), -jnp.inf, jnp.float32),
                      vmax_ref[...])
    va0 = jnp.where(j == 0, jnp.zeros((_ROWS, _SUB), jnp.uint32),
                    va_ref[...])

    def sub(k, carry, masked):
        vmax, va, a = carry
        bits = _threefry_bits(a)
        s = logits_ref[:, pl.ds(k * _SUB, _SUB)] * rtemp
        val = _gumbel_val(bits, s)
        if masked:
            val = jnp.where(a < bound, val, -jnp.inf)
        upd = val > vmax
        return (jnp.where(upd, val, vmax), jnp.where(upd, a, va),
                a + jnp.uint32(_SUB))

    @pl.when(j < _GRID - 1)
    def _():
        vmax1, va1, _ = jax.lax.fori_loop(
            0, _NSUB, lambda k, c: sub(k, c, False), (vmax0, va0, a0),
            unroll=2)
        vmax_ref[...] = vmax1
        va_ref[...] = va1

    bound = (rowoff + (_NCOLS + _KS1)).astype(jnp.uint32)

    @pl.when(j == _GRID - 1)
    def _():
        vmax1, va1, _ = jax.lax.fori_loop(
            0, _NSUB, lambda k, c: sub(k, c, True), (vmax0, va0, a0),
            unroll=2)
        rmax = jnp.max(vmax1, axis=1, keepdims=True)
        col = (va1.astype(jnp.int32) - _KS1) - rowoff
        cand = jnp.where(vmax1 == rmax, col, jnp.int32(2**31 - 1))
        idx_ref[...] = jnp.min(cand, axis=1, keepdims=True)


@functools.partial(jax.jit, static_argnames=("interpret",))
def _sample(logits, temperature, interpret=False):
    idx = pl.pallas_call(
        _body,
        grid=(_GRID,),
        in_specs=[
            pl.BlockSpec((_ROWS, _W), lambda j: (0, j)),
            pl.BlockSpec((_ROWS, 1), lambda j: (0, 0)),
        ],
        out_specs=pl.BlockSpec((_ROWS, 1), lambda j: (0, 0)),
        out_shape=jax.ShapeDtypeStruct((_ROWS, 1), jnp.int32),
        scratch_shapes=[
            pltpu.VMEM((_ROWS, _SUB), jnp.float32),
            pltpu.VMEM((_ROWS, _SUB), jnp.uint32),
        ],
        interpret=interpret,
    )(logits, temperature.reshape(_ROWS, 1))
    return idx[:, 0]


def kernel(logits, temperature):
    return _sample(logits, temperature)
